# Initial kernel scaffold; baseline (speedup 1.0000x reference)
import jax
import jax.numpy as jnp
from jax.experimental import pallas as pl


def kernel(tokens_id, weights):
    raise NotImplementedError("write your pallas kernel here")


# speedup vs baseline: 1.4592x; 1.4592x over previous
import functools

import jax
import jax.numpy as jnp
from jax import lax
from jax.experimental import pallas as pl
from jax.experimental.pallas import tpu as pltpu
from jax.experimental.pallas import tpu_sc as plsc

NUM_CORES = 2
NUM_SUBCORES = 16
NW = NUM_CORES * NUM_SUBCORES

IDX_MINOR = 128
KROWS = 8
BLOCK = KROWS * IDX_MINOR


def _make_gather(num_rows, dim, total_idx_rows):
    rows_per_w = total_idx_rows // NW
    nblocks = rows_per_w // KROWS
    mesh = plsc.VectorSubcoreMesh(
        core_axis_name="c", subcore_axis_name="s",
        num_cores=NUM_CORES, num_subcores=NUM_SUBCORES)

    @functools.partial(
        pl.kernel,
        out_type=jax.ShapeDtypeStruct((total_idx_rows * IDX_MINOR, dim),
                                      jnp.float32),
        mesh=mesh,
        scratch_types=[
            pltpu.VMEM((KROWS, IDX_MINOR), jnp.int32),
            pltpu.VMEM((BLOCK, dim), jnp.float32),
            pltpu.SemaphoreType.DMA,
        ],
        compiler_params=pltpu.CompilerParams(use_tc_tiling_on_sc=False),
    )
    def gather_kernel(table_hbm, idx_hbm, out_hbm, idx_v, rows_v, sem):
        wid = lax.axis_index("s") * NUM_CORES + lax.axis_index("c")
        row0 = wid * rows_per_w

        def body(blk, _):
            r0 = row0 + blk * KROWS
            pltpu.sync_copy(idx_hbm.at[pl.ds(r0, KROWS)], idx_v)
            descs = []
            for j in range(KROWS):
                descs.append(pltpu.async_copy(
                    table_hbm.at[idx_v.at[j]],
                    rows_v.at[pl.ds(j * IDX_MINOR, IDX_MINOR)],
                    sem))
            for d in descs:
                d.wait()
            pltpu.sync_copy(rows_v, out_hbm.at[pl.ds(r0 * IDX_MINOR, BLOCK)])
            return ()

        lax.fori_loop(0, nblocks, body, (), unroll=False)

    return gather_kernel


def kernel(tokens_id, weights):
    b, s = tokens_id.shape
    n, d = weights.shape
    total = b * s
    idx2d = tokens_id.astype(jnp.int32).reshape(total // IDX_MINOR, IDX_MINOR)
    fn = _make_gather(n, d, total // IDX_MINOR)
    out = fn(weights, idx2d)
    return out.reshape(b, s, d)


# speedup vs baseline: 1.5004x; 1.0282x over previous
import functools

import jax
import jax.numpy as jnp
from jax import lax
from jax.experimental import pallas as pl
from jax.experimental.pallas import tpu as pltpu
from jax.experimental.pallas import tpu_sc as plsc

NUM_CORES = 2
NUM_SUBCORES = 16
NW = NUM_CORES * NUM_SUBCORES

IDX_MINOR = 128
KROWS = 10
BLOCK = KROWS * IDX_MINOR


def _make_gather(num_rows, dim, total_idx_rows):
    rows_per_w = total_idx_rows // NW
    nblocks = rows_per_w // KROWS
    mesh = plsc.VectorSubcoreMesh(
        core_axis_name="c", subcore_axis_name="s",
        num_cores=NUM_CORES, num_subcores=NUM_SUBCORES)

    @functools.partial(
        pl.kernel,
        out_type=jax.ShapeDtypeStruct((total_idx_rows * IDX_MINOR, dim),
                                      jnp.float32),
        mesh=mesh,
        scratch_types=[
            pltpu.VMEM((rows_per_w, IDX_MINOR), jnp.int32),
            pltpu.VMEM((BLOCK, dim), jnp.float32),
            pltpu.VMEM((BLOCK, dim), jnp.float32),
            pltpu.SemaphoreType.DMA,
            pltpu.SemaphoreType.DMA,
            pltpu.SemaphoreType.DMA,
        ],
        compiler_params=pltpu.CompilerParams(use_tc_tiling_on_sc=False),
    )
    def gather_kernel(table_hbm, idx_hbm, out_hbm, idx_v, rows_a, rows_b,
                      gsem_a, gsem_b, osem):
        wid = lax.axis_index("s") * NUM_CORES + lax.axis_index("c")
        row0 = wid * rows_per_w

        pltpu.sync_copy(idx_hbm.at[pl.ds(row0, rows_per_w)], idx_v)

        def fire(g, buf, sem):
            for j in range(KROWS):
                pltpu.async_copy(
                    table_hbm.at[idx_v.at[g * KROWS + j]],
                    buf.at[pl.ds(j * IDX_MINOR, IDX_MINOR)],
                    sem)

        def drain_gathers(buf, sem):
            pltpu.make_async_copy(
                out_hbm.at[pl.ds(0, BLOCK)], buf, sem).wait()

        def write_out(g, buf):
            pltpu.async_copy(
                buf, out_hbm.at[pl.ds((row0 + g * KROWS) * IDX_MINOR, BLOCK)],
                osem).wait()

        fire(0, rows_a, gsem_a)
        fire(1, rows_b, gsem_b)

        def body(i, _):
            g = 2 * i
            drain_gathers(rows_a, gsem_a)
            write_out(g, rows_a)
            fire(g + 2, rows_a, gsem_a)
            drain_gathers(rows_b, gsem_b)
            write_out(g + 1, rows_b)
            fire(g + 3, rows_b, gsem_b)
            return ()

        lax.fori_loop(0, nblocks // 2 - 1, body, (), unroll=False)

        drain_gathers(rows_a, gsem_a)
        write_out(nblocks - 2, rows_a)
        drain_gathers(rows_b, gsem_b)
        write_out(nblocks - 1, rows_b)

    return gather_kernel


def kernel(tokens_id, weights):
    b, s = tokens_id.shape
    n, d = weights.shape
    total = b * s
    idx2d = tokens_id.astype(jnp.int32).reshape(total // IDX_MINOR, IDX_MINOR)
    fn = _make_gather(n, d, total // IDX_MINOR)
    out = fn(weights, idx2d)
    return out.reshape(b, s, d)
